# Initial kernel scaffold; baseline (speedup 1.0000x reference)
import functools

import jax
import jax.numpy as jnp
from jax import lax
from jax.experimental import pallas as pl
from jax.experimental.pallas import tpu as pltpu
from jax.experimental.pallas import tpu_sc as plsc

N = 10000
E = 320000
DREF = 145
D = 160
CNT_COL = 145
NPAD = 10240
NW = 32
K = 128
CHUNKS = 79
EPAD = NW * CHUNKS * K
ROWS_PER_TILE = NPAD // 16
RBLK = 512


def _sc_aggregate(hext, src, dst, zeros):
    mesh = plsc.VectorSubcoreMesh(core_axis_name="c", subcore_axis_name="s")

    @functools.partial(
        pl.kernel,
        mesh=mesh,
        out_type=jax.ShapeDtypeStruct((2, NPAD, D), jnp.float32),
        scratch_types=[
            pltpu.VMEM((CHUNKS, K), jnp.int32),
            pltpu.VMEM((CHUNKS, K), jnp.int32),
            pltpu.VMEM((K, D), jnp.float32),
            pltpu.VMEM_SHARED((NPAD, D), jnp.float32),
            pltpu.SemaphoreType.DMA,
        ],
    )
    def agg(h_hbm, src_hbm, dst_hbm, z_hbm, out_hbm, src_v, dst_v, rows_v, acc_sh, sem):
        cid = lax.axis_index("c")
        sid = lax.axis_index("s")
        wid = cid * 16 + sid
        r0 = sid * ROWS_PER_TILE
        pltpu.sync_copy(z_hbm.at[pl.ds(r0, ROWS_PER_TILE)],
                        acc_sh.at[pl.ds(r0, ROWS_PER_TILE)])
        pltpu.sync_copy(src_hbm.at[wid], src_v)
        pltpu.sync_copy(dst_hbm.at[wid], dst_v)
        plsc.subcore_barrier()

        def body(j, carry):
            pltpu.async_copy(h_hbm.at[src_v.at[j]], rows_v, sem).wait()
            pltpu.sync_copy(rows_v, acc_sh.at[dst_v.at[j]], add=True)
            return carry

        lax.fori_loop(0, CHUNKS, body, 0)
        plsc.subcore_barrier()
        pltpu.sync_copy(acc_sh.at[pl.ds(r0, ROWS_PER_TILE)],
                        out_hbm.at[cid, pl.ds(r0, ROWS_PER_TILE)])

    return agg(hext, src, dst, zeros)


def _tc_layer_body(acc_ref, h_ref, lm_ref, vlt, wlt, vut, wut, bl, bu, o_ref):
    a = acc_ref[0] + acc_ref[1]
    inv = 1.0 / jnp.maximum(a[:, CNT_COL:CNT_COL + 1], 1.0)
    p = a * inv
    h = h_ref[...]
    f32 = jnp.float32
    lab = (jnp.dot(h, vlt[...], preferred_element_type=f32)
           + jnp.dot(p, wlt[...], preferred_element_type=f32) + bl[...])
    unl = (jnp.dot(h, vut[...], preferred_element_type=f32)
           + jnp.dot(p, wut[...], preferred_element_type=f32) + bu[...])
    lm = lm_ref[...]
    o_ref[...] = jnp.maximum(lm * lab + (1.0 - lm) * unl, 0.0)


def _tc_final_body(acc_ref, h_ref, lm_ref, vlt, wlt, vut, wut, bl, bu,
                   cwt, cb, o_ref):
    a = acc_ref[0] + acc_ref[1]
    inv = 1.0 / jnp.maximum(a[:, CNT_COL:CNT_COL + 1], 1.0)
    p = a * inv
    h = h_ref[...]
    f32 = jnp.float32
    lab = (jnp.dot(h, vlt[...], preferred_element_type=f32)
           + jnp.dot(p, wlt[...], preferred_element_type=f32) + bl[...])
    unl = (jnp.dot(h, vut[...], preferred_element_type=f32)
           + jnp.dot(p, wut[...], preferred_element_type=f32) + bu[...])
    lm = lm_ref[...]
    h1 = jnp.maximum(lm * lab + (1.0 - lm) * unl, 0.0)
    logits = jnp.dot(h1, cwt[...], preferred_element_type=f32) + cb[...]
    m = jnp.max(logits, axis=1, keepdims=True)
    s = logits - m
    o_ref[...] = s - jnp.log(jnp.sum(jnp.exp(s), axis=1, keepdims=True))


def _row_spec(r, c):
    return pl.BlockSpec((r, c), lambda i: (i, 0))


def _rep_spec(r, c):
    return pl.BlockSpec((r, c), lambda i: (0, 0))


def _tc_layer(acc, hext, lmf, vlt, wlt, vut, wut, bl, bu):
    grid = (NPAD // RBLK,)
    return pl.pallas_call(
        _tc_layer_body,
        grid=grid,
        in_specs=[
            pl.BlockSpec((2, RBLK, D), lambda i: (0, i, 0)),
            _row_spec(RBLK, D),
            _row_spec(RBLK, 1),
            _rep_spec(D, D), _rep_spec(D, D), _rep_spec(D, D), _rep_spec(D, D),
            _rep_spec(1, D), _rep_spec(1, D),
        ],
        out_specs=_row_spec(RBLK, D),
        out_shape=jax.ShapeDtypeStruct((NPAD, D), jnp.float32),
    )(acc, hext, lmf, vlt, wlt, vut, wut, bl, bu)


def _tc_final(acc, hext, lmf, vlt, wlt, vut, wut, bl, bu, cwt, cb):
    grid = (NPAD // RBLK,)
    nc = cwt.shape[1]
    return pl.pallas_call(
        _tc_final_body,
        grid=grid,
        in_specs=[
            pl.BlockSpec((2, RBLK, D), lambda i: (0, i, 0)),
            _row_spec(RBLK, D),
            _row_spec(RBLK, 1),
            _rep_spec(D, D), _rep_spec(D, D), _rep_spec(D, D), _rep_spec(D, D),
            _rep_spec(1, D), _rep_spec(1, D),
            _rep_spec(D, nc), _rep_spec(1, nc),
        ],
        out_specs=_row_spec(RBLK, nc),
        out_shape=jax.ShapeDtypeStruct((NPAD, nc), jnp.float32),
    )(acc, hext, lmf, vlt, wlt, vut, wut, bl, bu, cwt, cb)


def _pad_wt(w):
    return jnp.zeros((D, D), jnp.float32).at[:DREF, :DREF].set(w.T)


def _pad_bias(b1, b2):
    return jnp.zeros((1, D), jnp.float32).at[0, :DREF].set(b1 + b2).at[0, CNT_COL].set(1.0)


def kernel(x, edge_index, labelmask,
           l0_VLw, l0_VLb, l0_WLw, l0_WLb, l0_VUw, l0_VUb, l0_WUw, l0_WUb,
           l1_VLw, l1_VLb, l1_WLw, l1_WLb, l1_VUw, l1_VUb, l1_WUw, l1_WUb,
           Cw, Cb):
    hext0 = jnp.zeros((NPAD, D), jnp.float32)
    hext0 = hext0.at[:N, :DREF].set(x).at[:N, CNT_COL].set(1.0)
    pad = jnp.full((EPAD - E,), N, jnp.int32)
    src = jnp.concatenate([edge_index[0], pad]).reshape(NW, CHUNKS, K)
    dst = jnp.concatenate([edge_index[1], pad]).reshape(NW, CHUNKS, K)
    zeros = jnp.zeros((NPAD, D), jnp.float32)
    lmf = jnp.zeros((NPAD, 1), jnp.float32).at[:N, 0].set(labelmask.astype(jnp.float32))

    l0 = (_pad_wt(l0_VLw), _pad_wt(l0_WLw), _pad_wt(l0_VUw), _pad_wt(l0_WUw),
          _pad_bias(l0_VLb, l0_WLb), _pad_bias(l0_VUb, l0_WUb))
    l1 = (_pad_wt(l1_VLw), _pad_wt(l1_WLw), _pad_wt(l1_VUw), _pad_wt(l1_WUw),
          _pad_bias(l1_VLb, l1_WLb), _pad_bias(l1_VUb, l1_WUb))
    nc = Cw.shape[0]
    cwt = jnp.zeros((D, nc), jnp.float32).at[:DREF].set(Cw.T)
    cb = Cb.reshape(1, nc)

    acc0 = _sc_aggregate(hext0, src, dst, zeros)
    hext1 = _tc_layer(acc0, hext0, lmf, *l0)
    acc1 = _sc_aggregate(hext1, src, dst, zeros)
    out = _tc_final(acc1, hext1, lmf, *l1, cwt, cb)
    return out[:N]


# speedup vs baseline: 6.0179x; 6.0179x over previous
import functools

import jax
import jax.numpy as jnp
from jax import lax
from jax.experimental import pallas as pl
from jax.experimental.pallas import tpu as pltpu
from jax.experimental.pallas import tpu_sc as plsc

N = 10000
E = 320000
DREF = 145
D = 160
DH = D // 2
CNT_COL = 145
NPAD = 10240
NT = 16
K = 128
NCH = E // K
CHB = NCH // NT
NEXTRA = NCH - CHB * NT
ROWS_PER_TILE = NPAD // NT
RBLK = 512


@functools.cache
def _make_sc_aggregate():
    mesh = plsc.VectorSubcoreMesh(core_axis_name="c", subcore_axis_name="s")

    @functools.partial(
        pl.kernel,
        mesh=mesh,
        compiler_params=pltpu.CompilerParams(use_tc_tiling_on_sc=False),
        out_type=jax.ShapeDtypeStruct((2, NPAD, DH), jnp.float32),
        scratch_types=[
            pltpu.VMEM((CHB + 1, K), jnp.int32),
            pltpu.VMEM((CHB + 1, K), jnp.int32),
            pltpu.VMEM((K, DH), jnp.float32),
            pltpu.VMEM_SHARED((NPAD, DH), jnp.float32),
            pltpu.SemaphoreType.DMA,
        ],
    )
    def agg(h_hbm, ei_hbm, z_hbm, out_hbm, src_v, dst_v, rows_v, acc_sh, sem):
        cid = lax.axis_index("c")
        sid = lax.axis_index("s")
        r0 = sid * ROWS_PER_TILE
        pltpu.sync_copy(z_hbm.at[pl.ds(r0, ROWS_PER_TILE)],
                        acc_sh.at[pl.ds(r0, ROWS_PER_TILE)])
        pltpu.sync_copy(ei_hbm.at[0, pl.ds(sid * CHB, CHB)],
                        src_v.at[pl.ds(0, CHB)])
        pltpu.sync_copy(ei_hbm.at[1, pl.ds(sid * CHB, CHB)],
                        dst_v.at[pl.ds(0, CHB)])

        @pl.when(sid < NEXTRA)
        def _():
            pltpu.sync_copy(ei_hbm.at[0, pl.ds(NT * CHB + sid, 1)],
                            src_v.at[pl.ds(CHB, 1)])
            pltpu.sync_copy(ei_hbm.at[1, pl.ds(NT * CHB + sid, 1)],
                            dst_v.at[pl.ds(CHB, 1)])

        plsc.subcore_barrier()

        def step(j):
            pltpu.async_copy(h_hbm.at[cid].at[src_v.at[j]], rows_v, sem).wait()
            pltpu.sync_copy(rows_v, acc_sh.at[dst_v.at[j]], add=True)

        def body(j, carry):
            step(j)
            return carry

        lax.fori_loop(0, CHB, body, 0)

        @pl.when(sid < NEXTRA)
        def _():
            step(CHB)

        plsc.subcore_barrier()
        pltpu.sync_copy(acc_sh.at[pl.ds(r0, ROWS_PER_TILE)],
                        out_hbm.at[cid, pl.ds(r0, ROWS_PER_TILE)])

    return agg


def _sc_aggregate(hsplit, ei, zeros):
    return _make_sc_aggregate()(hsplit, ei, zeros)


def _mean_and_h(acc_ref, h_ref):
    a = jnp.concatenate([acc_ref[0], acc_ref[1]], axis=1)
    inv = 1.0 / jnp.maximum(a[:, CNT_COL:CNT_COL + 1], 1.0)
    p = a * inv
    h = jnp.concatenate([h_ref[0], h_ref[1]], axis=1)
    return p, h


def _masked_linear(p, h, lm_ref, vlt, wlt, vut, wut, bl, bu):
    f32 = jnp.float32
    lab = (jnp.dot(h, vlt[...], preferred_element_type=f32)
           + jnp.dot(p, wlt[...], preferred_element_type=f32) + bl[...])
    unl = (jnp.dot(h, vut[...], preferred_element_type=f32)
           + jnp.dot(p, wut[...], preferred_element_type=f32) + bu[...])
    lm = lm_ref[...]
    return jnp.maximum(lm * lab + (1.0 - lm) * unl, 0.0)


def _tc_layer_body(acc_ref, h_ref, lm_ref, vlt, wlt, vut, wut, bl, bu, o_ref):
    p, h = _mean_and_h(acc_ref, h_ref)
    res = _masked_linear(p, h, lm_ref, vlt, wlt, vut, wut, bl, bu)
    o_ref[0] = res[:, :DH]
    o_ref[1] = res[:, DH:]


def _tc_final_body(acc_ref, h_ref, lm_ref, vlt, wlt, vut, wut, bl, bu,
                   cwt, cb, o_ref):
    p, h = _mean_and_h(acc_ref, h_ref)
    h1 = _masked_linear(p, h, lm_ref, vlt, wlt, vut, wut, bl, bu)
    logits = jnp.dot(h1, cwt[...], preferred_element_type=jnp.float32) + cb[...]
    m = jnp.max(logits, axis=1, keepdims=True)
    s = logits - m
    o_ref[...] = s - jnp.log(jnp.sum(jnp.exp(s), axis=1, keepdims=True))


def _row_spec(r, c):
    return pl.BlockSpec((r, c), lambda i: (i, 0))


def _rep_spec(r, c):
    return pl.BlockSpec((r, c), lambda i: (0, 0))


_SPLIT_SPEC = pl.BlockSpec((2, RBLK, DH), lambda i: (0, i, 0))


def _tc_layer(acc, hsplit, lmf, vlt, wlt, vut, wut, bl, bu):
    grid = (NPAD // RBLK,)
    return pl.pallas_call(
        _tc_layer_body,
        grid=grid,
        in_specs=[
            _SPLIT_SPEC,
            _SPLIT_SPEC,
            _row_spec(RBLK, 1),
            _rep_spec(D, D), _rep_spec(D, D), _rep_spec(D, D), _rep_spec(D, D),
            _rep_spec(1, D), _rep_spec(1, D),
        ],
        out_specs=_SPLIT_SPEC,
        out_shape=jax.ShapeDtypeStruct((2, NPAD, DH), jnp.float32),
    )(acc, hsplit, lmf, vlt, wlt, vut, wut, bl, bu)


def _tc_final(acc, hsplit, lmf, vlt, wlt, vut, wut, bl, bu, cwt, cb):
    grid = (NPAD // RBLK,)
    nc = cwt.shape[1]
    return pl.pallas_call(
        _tc_final_body,
        grid=grid,
        in_specs=[
            _SPLIT_SPEC,
            _SPLIT_SPEC,
            _row_spec(RBLK, 1),
            _rep_spec(D, D), _rep_spec(D, D), _rep_spec(D, D), _rep_spec(D, D),
            _rep_spec(1, D), _rep_spec(1, D),
            _rep_spec(D, nc), _rep_spec(1, nc),
        ],
        out_specs=_row_spec(RBLK, nc),
        out_shape=jax.ShapeDtypeStruct((NPAD, nc), jnp.float32),
    )(acc, hsplit, lmf, vlt, wlt, vut, wut, bl, bu, cwt, cb)


def _pad_wt(w):
    return jnp.zeros((D, D), jnp.float32).at[:DREF, :DREF].set(w.T)


def _pad_bias(b1, b2):
    return jnp.zeros((1, D), jnp.float32).at[0, :DREF].set(b1 + b2).at[0, CNT_COL].set(1.0)


def kernel(x, edge_index, labelmask,
           l0_VLw, l0_VLb, l0_WLw, l0_WLb, l0_VUw, l0_VUb, l0_WUw, l0_WUb,
           l1_VLw, l1_VLb, l1_WLw, l1_WLb, l1_VUw, l1_VUb, l1_WUw, l1_WUb,
           Cw, Cb):
    hs0 = jnp.zeros((2, NPAD, DH), jnp.float32)
    hs0 = (hs0.at[0, :N, :].set(x[:, :DH])
              .at[1, :N, :DREF - DH].set(x[:, DH:DREF])
              .at[1, :N, CNT_COL - DH].set(1.0))
    ei3 = edge_index.reshape(2, NCH, K)
    zeros = jnp.zeros((NPAD, DH), jnp.float32)
    lmf = jnp.zeros((NPAD, 1), jnp.float32).at[:N, 0].set(labelmask.astype(jnp.float32))

    l0 = (_pad_wt(l0_VLw), _pad_wt(l0_WLw), _pad_wt(l0_VUw), _pad_wt(l0_WUw),
          _pad_bias(l0_VLb, l0_WLb), _pad_bias(l0_VUb, l0_WUb))
    l1 = (_pad_wt(l1_VLw), _pad_wt(l1_WLw), _pad_wt(l1_VUw), _pad_wt(l1_WUw),
          _pad_bias(l1_VLb, l1_WLb), _pad_bias(l1_VUb, l1_WUb))
    nc = Cw.shape[0]
    cwt = jnp.zeros((D, nc), jnp.float32).at[:DREF].set(Cw.T)
    cb = Cb.reshape(1, nc)

    acc0 = _sc_aggregate(hs0, ei3, zeros)
    hs1 = _tc_layer(acc0, hs0, lmf, *l0)
    acc1 = _sc_aggregate(hs1, ei3, zeros)
    out = _tc_final(acc1, hs1, lmf, *l1, cwt, cb)
    return out[:N]


# speedup vs baseline: 7.3939x; 1.2287x over previous
import functools

import jax
import jax.numpy as jnp
from jax import lax
from jax.experimental import pallas as pl
from jax.experimental.pallas import tpu as pltpu
from jax.experimental.pallas import tpu_sc as plsc

N = 10000
E = 320000
DREF = 145
D = 160
DH = D // 2
CNT_COL = 145
NPAD = 10240
NT = 16
K = 128
NCH = E // K
CHB = NCH // NT
NEXTRA = NCH - CHB * NT
ROWS_PER_TILE = NPAD // NT
RBLK = 512


@functools.cache
def _make_sc_aggregate():
    mesh = plsc.VectorSubcoreMesh(core_axis_name="c", subcore_axis_name="s")

    @functools.partial(
        pl.kernel,
        mesh=mesh,
        compiler_params=pltpu.CompilerParams(use_tc_tiling_on_sc=False),
        out_type=jax.ShapeDtypeStruct((2, NPAD, DH), jnp.float32),
        scratch_types=[
            pltpu.VMEM((CHB + 1, K), jnp.int32),
            pltpu.VMEM((CHB + 1, K), jnp.int32),
            pltpu.VMEM((K, DH), jnp.float32),
            pltpu.VMEM((K, DH), jnp.float32),
            pltpu.VMEM_SHARED((NPAD, DH), jnp.float32),
            pltpu.SemaphoreType.DMA,
        ],
    )
    def agg(h_hbm, ei_hbm, z_hbm, out_hbm, src_v, dst_v, rows0_v, rows1_v,
            acc_sh, sem):
        cid = lax.axis_index("c")
        sid = lax.axis_index("s")
        r0 = sid * ROWS_PER_TILE
        pltpu.sync_copy(z_hbm.at[pl.ds(r0, ROWS_PER_TILE)],
                        acc_sh.at[pl.ds(r0, ROWS_PER_TILE)])
        pltpu.sync_copy(ei_hbm.at[0, pl.ds(sid * CHB, CHB)],
                        src_v.at[pl.ds(0, CHB)])
        pltpu.sync_copy(ei_hbm.at[1, pl.ds(sid * CHB, CHB)],
                        dst_v.at[pl.ds(0, CHB)])

        @pl.when(sid < NEXTRA)
        def _():
            pltpu.sync_copy(ei_hbm.at[0, pl.ds(NT * CHB + sid, 1)],
                            src_v.at[pl.ds(CHB, 1)])
            pltpu.sync_copy(ei_hbm.at[1, pl.ds(NT * CHB + sid, 1)],
                            dst_v.at[pl.ds(CHB, 1)])

        plsc.subcore_barrier()

        def gather(j, buf):
            return pltpu.make_async_copy(h_hbm.at[cid].at[src_v.at[j]], buf, sem)

        gather(0, rows0_v).start()

        def body(t, carry):
            j0 = 2 * t
            j1 = j0 + 1
            gather(j0, rows0_v).wait()
            gather(j1, rows1_v).start()
            pltpu.sync_copy(rows0_v, acc_sh.at[dst_v.at[j0]], add=True)
            gather(j0, rows0_v).wait()

            @pl.when(j1 + 1 < CHB)
            def _():
                gather(j1 + 1, rows0_v).start()

            pltpu.sync_copy(rows1_v, acc_sh.at[dst_v.at[j1]], add=True)
            return carry

        lax.fori_loop(0, CHB // 2, body, 0)

        @pl.when(sid < NEXTRA)
        def _():
            gather(CHB, rows0_v).start()
            gather(CHB, rows0_v).wait()
            pltpu.sync_copy(rows0_v, acc_sh.at[dst_v.at[CHB]], add=True)

        plsc.subcore_barrier()
        pltpu.sync_copy(acc_sh.at[pl.ds(r0, ROWS_PER_TILE)],
                        out_hbm.at[cid, pl.ds(r0, ROWS_PER_TILE)])

    return agg


def _sc_aggregate(hsplit, ei, zeros):
    return _make_sc_aggregate()(hsplit, ei, zeros)


def _mean_and_h(acc_ref, h_ref):
    a = jnp.concatenate([acc_ref[0], acc_ref[1]], axis=1)
    inv = 1.0 / jnp.maximum(a[:, CNT_COL:CNT_COL + 1], 1.0)
    p = a * inv
    h = jnp.concatenate([h_ref[0], h_ref[1]], axis=1)
    return p, h


def _masked_linear(p, h, lm_ref, vlt, wlt, vut, wut, bl, bu):
    f32 = jnp.float32
    lab = (jnp.dot(h, vlt[...], preferred_element_type=f32)
           + jnp.dot(p, wlt[...], preferred_element_type=f32) + bl[...])
    unl = (jnp.dot(h, vut[...], preferred_element_type=f32)
           + jnp.dot(p, wut[...], preferred_element_type=f32) + bu[...])
    lm = lm_ref[...]
    return jnp.maximum(lm * lab + (1.0 - lm) * unl, 0.0)


def _tc_layer_body(acc_ref, h_ref, lm_ref, vlt, wlt, vut, wut, bl, bu, o_ref):
    p, h = _mean_and_h(acc_ref, h_ref)
    res = _masked_linear(p, h, lm_ref, vlt, wlt, vut, wut, bl, bu)
    o_ref[0] = res[:, :DH]
    o_ref[1] = res[:, DH:]


def _tc_final_body(acc_ref, h_ref, lm_ref, vlt, wlt, vut, wut, bl, bu,
                   cwt, cb, o_ref):
    p, h = _mean_and_h(acc_ref, h_ref)
    h1 = _masked_linear(p, h, lm_ref, vlt, wlt, vut, wut, bl, bu)
    logits = jnp.dot(h1, cwt[...], preferred_element_type=jnp.float32) + cb[...]
    m = jnp.max(logits, axis=1, keepdims=True)
    s = logits - m
    o_ref[...] = s - jnp.log(jnp.sum(jnp.exp(s), axis=1, keepdims=True))


def _row_spec(r, c):
    return pl.BlockSpec((r, c), lambda i: (i, 0))


def _rep_spec(r, c):
    return pl.BlockSpec((r, c), lambda i: (0, 0))


_SPLIT_SPEC = pl.BlockSpec((2, RBLK, DH), lambda i: (0, i, 0))


def _tc_layer(acc, hsplit, lmf, vlt, wlt, vut, wut, bl, bu):
    grid = (NPAD // RBLK,)
    return pl.pallas_call(
        _tc_layer_body,
        grid=grid,
        in_specs=[
            _SPLIT_SPEC,
            _SPLIT_SPEC,
            _row_spec(RBLK, 1),
            _rep_spec(D, D), _rep_spec(D, D), _rep_spec(D, D), _rep_spec(D, D),
            _rep_spec(1, D), _rep_spec(1, D),
        ],
        out_specs=_SPLIT_SPEC,
        out_shape=jax.ShapeDtypeStruct((2, NPAD, DH), jnp.float32),
    )(acc, hsplit, lmf, vlt, wlt, vut, wut, bl, bu)


def _tc_final(acc, hsplit, lmf, vlt, wlt, vut, wut, bl, bu, cwt, cb):
    grid = (NPAD // RBLK,)
    nc = cwt.shape[1]
    return pl.pallas_call(
        _tc_final_body,
        grid=grid,
        in_specs=[
            _SPLIT_SPEC,
            _SPLIT_SPEC,
            _row_spec(RBLK, 1),
            _rep_spec(D, D), _rep_spec(D, D), _rep_spec(D, D), _rep_spec(D, D),
            _rep_spec(1, D), _rep_spec(1, D),
            _rep_spec(D, nc), _rep_spec(1, nc),
        ],
        out_specs=_row_spec(RBLK, nc),
        out_shape=jax.ShapeDtypeStruct((NPAD, nc), jnp.float32),
    )(acc, hsplit, lmf, vlt, wlt, vut, wut, bl, bu, cwt, cb)


def _pad_wt(w):
    return jnp.zeros((D, D), jnp.float32).at[:DREF, :DREF].set(w.T)


def _pad_bias(b1, b2):
    return jnp.zeros((1, D), jnp.float32).at[0, :DREF].set(b1 + b2).at[0, CNT_COL].set(1.0)


def kernel(x, edge_index, labelmask,
           l0_VLw, l0_VLb, l0_WLw, l0_WLb, l0_VUw, l0_VUb, l0_WUw, l0_WUb,
           l1_VLw, l1_VLb, l1_WLw, l1_WLb, l1_VUw, l1_VUb, l1_WUw, l1_WUb,
           Cw, Cb):
    hs0 = jnp.zeros((2, NPAD, DH), jnp.float32)
    hs0 = (hs0.at[0, :N, :].set(x[:, :DH])
              .at[1, :N, :DREF - DH].set(x[:, DH:DREF])
              .at[1, :N, CNT_COL - DH].set(1.0))
    ei3 = edge_index.reshape(2, NCH, K)
    zeros = jnp.zeros((NPAD, DH), jnp.float32)
    lmf = jnp.zeros((NPAD, 1), jnp.float32).at[:N, 0].set(labelmask.astype(jnp.float32))

    l0 = (_pad_wt(l0_VLw), _pad_wt(l0_WLw), _pad_wt(l0_VUw), _pad_wt(l0_WUw),
          _pad_bias(l0_VLb, l0_WLb), _pad_bias(l0_VUb, l0_WUb))
    l1 = (_pad_wt(l1_VLw), _pad_wt(l1_WLw), _pad_wt(l1_VUw), _pad_wt(l1_WUw),
          _pad_bias(l1_VLb, l1_WLb), _pad_bias(l1_VUb, l1_WUb))
    nc = Cw.shape[0]
    cwt = jnp.zeros((D, nc), jnp.float32).at[:DREF].set(Cw.T)
    cb = Cb.reshape(1, nc)

    acc0 = _sc_aggregate(hs0, ei3, zeros)
    hs1 = _tc_layer(acc0, hs0, lmf, *l0)
    acc1 = _sc_aggregate(hs1, ei3, zeros)
    out = _tc_final(acc1, hs1, lmf, *l1, cwt, cb)
    return out[:N]


# speedup vs baseline: 9.2370x; 1.2493x over previous
import functools

import jax
import jax.numpy as jnp
from jax import lax
from jax.experimental import pallas as pl
from jax.experimental.pallas import tpu as pltpu
from jax.experimental.pallas import tpu_sc as plsc

N = 10000
E = 320000
DREF = 145
D = 160
DH = D // 2
CNT_COL = 145
NPAD = 10240
NT = 16
K = 128
NCH = E // K
CHB = NCH // NT
NEXTRA = NCH - CHB * NT
ROWS_PER_TILE = NPAD // NT
RBLK = 512


@functools.cache
def _make_sc_aggregate():
    mesh = plsc.VectorSubcoreMesh(core_axis_name="c", subcore_axis_name="s")

    @functools.partial(
        pl.kernel,
        mesh=mesh,
        compiler_params=pltpu.CompilerParams(use_tc_tiling_on_sc=False),
        out_type=jax.ShapeDtypeStruct((2, NPAD, DH), jnp.float32),
        scratch_types=[
            pltpu.VMEM((CHB + 1, K), jnp.int32),
            pltpu.VMEM((CHB + 1, K), jnp.int32),
            pltpu.VMEM((K, DH), jnp.float32),
            pltpu.VMEM((K, DH), jnp.float32),
            pltpu.VMEM((K, DH), jnp.float32),
            pltpu.VMEM_SHARED((NPAD, DH), jnp.float32),
            pltpu.SemaphoreType.DMA,
            pltpu.SemaphoreType.DMA,
        ],
    )
    def agg(h_hbm, ei_hbm, z_hbm, out_hbm, src_v, dst_v, rows0_v, rows1_v,
            rows2_v, acc_sh, gsem, ssem):
        cid = lax.axis_index("c")
        sid = lax.axis_index("s")
        r0 = sid * ROWS_PER_TILE
        pltpu.sync_copy(z_hbm.at[pl.ds(r0, ROWS_PER_TILE)],
                        acc_sh.at[pl.ds(r0, ROWS_PER_TILE)])
        pltpu.sync_copy(ei_hbm.at[0, pl.ds(sid * CHB, CHB)],
                        src_v.at[pl.ds(0, CHB)])
        pltpu.sync_copy(ei_hbm.at[1, pl.ds(sid * CHB, CHB)],
                        dst_v.at[pl.ds(0, CHB)])

        @pl.when(sid < NEXTRA)
        def _():
            pltpu.sync_copy(ei_hbm.at[0, pl.ds(NT * CHB + sid, 1)],
                            src_v.at[pl.ds(CHB, 1)])
            pltpu.sync_copy(ei_hbm.at[1, pl.ds(NT * CHB + sid, 1)],
                            dst_v.at[pl.ds(CHB, 1)])

        plsc.subcore_barrier()

        bufs = (rows0_v, rows1_v, rows2_v)

        def gather(j, buf):
            return pltpu.make_async_copy(h_hbm.at[cid].at[src_v.at[j]], buf, gsem)

        def scatter(j, buf):
            return pltpu.make_async_copy(buf, acc_sh.at[dst_v.at[j]], ssem)

        gather(0, rows0_v).start()
        gather(1, rows1_v).start()

        def body(t, carry):
            for r in range(3):
                j = 3 * t + r
                gather(j, bufs[r]).wait()

                @pl.when(j >= 1)
                def _():
                    scatter(j, bufs[r]).wait()

                @pl.when(j + 2 < CHB)
                def _():
                    gather(j + 2, bufs[(r + 2) % 3]).start()

                scatter(j, bufs[r]).start(add=True)
            return carry

        lax.fori_loop(0, CHB // 3, body, 0)
        scatter(0, rows0_v).wait()

        @pl.when(sid < NEXTRA)
        def _():
            gather(CHB, rows0_v).start()
            gather(CHB, rows0_v).wait()
            pltpu.sync_copy(rows0_v, acc_sh.at[dst_v.at[CHB]], add=True)

        plsc.subcore_barrier()
        pltpu.sync_copy(acc_sh.at[pl.ds(r0, ROWS_PER_TILE)],
                        out_hbm.at[cid, pl.ds(r0, ROWS_PER_TILE)])

    return agg


def _sc_aggregate(hsplit, ei, zeros):
    return _make_sc_aggregate()(hsplit, ei, zeros)


def _mean_and_h(acc_ref, h_ref):
    a = jnp.concatenate([acc_ref[0], acc_ref[1]], axis=1)
    inv = 1.0 / jnp.maximum(a[:, CNT_COL:CNT_COL + 1], 1.0)
    p = a * inv
    h = jnp.concatenate([h_ref[0], h_ref[1]], axis=1)
    return p, h


def _masked_linear(p, h, lm_ref, vlt, wlt, vut, wut, bl, bu):
    f32 = jnp.float32
    lab = (jnp.dot(h, vlt[...], preferred_element_type=f32)
           + jnp.dot(p, wlt[...], preferred_element_type=f32) + bl[...])
    unl = (jnp.dot(h, vut[...], preferred_element_type=f32)
           + jnp.dot(p, wut[...], preferred_element_type=f32) + bu[...])
    lm = lm_ref[...]
    return jnp.maximum(lm * lab + (1.0 - lm) * unl, 0.0)


def _tc_layer_body(acc_ref, h_ref, lm_ref, vlt, wlt, vut, wut, bl, bu, o_ref):
    p, h = _mean_and_h(acc_ref, h_ref)
    res = _masked_linear(p, h, lm_ref, vlt, wlt, vut, wut, bl, bu)
    o_ref[0] = res[:, :DH]
    o_ref[1] = res[:, DH:]


def _tc_final_body(acc_ref, h_ref, lm_ref, vlt, wlt, vut, wut, bl, bu,
                   cwt, cb, o_ref):
    p, h = _mean_and_h(acc_ref, h_ref)
    h1 = _masked_linear(p, h, lm_ref, vlt, wlt, vut, wut, bl, bu)
    logits = jnp.dot(h1, cwt[...], preferred_element_type=jnp.float32) + cb[...]
    m = jnp.max(logits, axis=1, keepdims=True)
    s = logits - m
    o_ref[...] = s - jnp.log(jnp.sum(jnp.exp(s), axis=1, keepdims=True))


def _row_spec(r, c):
    return pl.BlockSpec((r, c), lambda i: (i, 0))


def _rep_spec(r, c):
    return pl.BlockSpec((r, c), lambda i: (0, 0))


_SPLIT_SPEC = pl.BlockSpec((2, RBLK, DH), lambda i: (0, i, 0))


def _tc_layer(acc, hsplit, lmf, vlt, wlt, vut, wut, bl, bu):
    grid = (NPAD // RBLK,)
    return pl.pallas_call(
        _tc_layer_body,
        grid=grid,
        in_specs=[
            _SPLIT_SPEC,
            _SPLIT_SPEC,
            _row_spec(RBLK, 1),
            _rep_spec(D, D), _rep_spec(D, D), _rep_spec(D, D), _rep_spec(D, D),
            _rep_spec(1, D), _rep_spec(1, D),
        ],
        out_specs=_SPLIT_SPEC,
        out_shape=jax.ShapeDtypeStruct((2, NPAD, DH), jnp.float32),
    )(acc, hsplit, lmf, vlt, wlt, vut, wut, bl, bu)


def _tc_final(acc, hsplit, lmf, vlt, wlt, vut, wut, bl, bu, cwt, cb):
    grid = (NPAD // RBLK,)
    nc = cwt.shape[1]
    return pl.pallas_call(
        _tc_final_body,
        grid=grid,
        in_specs=[
            _SPLIT_SPEC,
            _SPLIT_SPEC,
            _row_spec(RBLK, 1),
            _rep_spec(D, D), _rep_spec(D, D), _rep_spec(D, D), _rep_spec(D, D),
            _rep_spec(1, D), _rep_spec(1, D),
            _rep_spec(D, nc), _rep_spec(1, nc),
        ],
        out_specs=_row_spec(RBLK, nc),
        out_shape=jax.ShapeDtypeStruct((NPAD, nc), jnp.float32),
    )(acc, hsplit, lmf, vlt, wlt, vut, wut, bl, bu, cwt, cb)


def _pad_wt(w):
    return jnp.zeros((D, D), jnp.float32).at[:DREF, :DREF].set(w.T)


def _pad_bias(b1, b2):
    return jnp.zeros((1, D), jnp.float32).at[0, :DREF].set(b1 + b2).at[0, CNT_COL].set(1.0)


def kernel(x, edge_index, labelmask,
           l0_VLw, l0_VLb, l0_WLw, l0_WLb, l0_VUw, l0_VUb, l0_WUw, l0_WUb,
           l1_VLw, l1_VLb, l1_WLw, l1_WLb, l1_VUw, l1_VUb, l1_WUw, l1_WUb,
           Cw, Cb):
    hs0 = jnp.zeros((2, NPAD, DH), jnp.float32)
    hs0 = (hs0.at[0, :N, :].set(x[:, :DH])
              .at[1, :N, :DREF - DH].set(x[:, DH:DREF])
              .at[1, :N, CNT_COL - DH].set(1.0))
    ei3 = edge_index.reshape(2, NCH, K)
    zeros = jnp.zeros((NPAD, DH), jnp.float32)
    lmf = jnp.zeros((NPAD, 1), jnp.float32).at[:N, 0].set(labelmask.astype(jnp.float32))

    l0 = (_pad_wt(l0_VLw), _pad_wt(l0_WLw), _pad_wt(l0_VUw), _pad_wt(l0_WUw),
          _pad_bias(l0_VLb, l0_WLb), _pad_bias(l0_VUb, l0_WUb))
    l1 = (_pad_wt(l1_VLw), _pad_wt(l1_WLw), _pad_wt(l1_VUw), _pad_wt(l1_WUw),
          _pad_bias(l1_VLb, l1_WLb), _pad_bias(l1_VUb, l1_WUb))
    nc = Cw.shape[0]
    cwt = jnp.zeros((D, nc), jnp.float32).at[:DREF].set(Cw.T)
    cb = Cb.reshape(1, nc)

    acc0 = _sc_aggregate(hs0, ei3, zeros)
    hs1 = _tc_layer(acc0, hs0, lmf, *l0)
    acc1 = _sc_aggregate(hs1, ei3, zeros)
    out = _tc_final(acc1, hs1, lmf, *l1, cwt, cb)
    return out[:N]


# speedup vs baseline: 10.1000x; 1.0934x over previous
import functools

import jax
import jax.numpy as jnp
from jax import lax
from jax.experimental import pallas as pl
from jax.experimental.pallas import tpu as pltpu
from jax.experimental.pallas import tpu_sc as plsc

N = 10000
E = 320000
DREF = 145
D = 160
DH = D // 2
CNT_COL = 145
NPAD = 10240
NT = 16
K = 128
NCH = E // K
CHB = NCH // NT
NEXTRA = NCH - CHB * NT
ROWS_PER_TILE = NPAD // NT
RBLK = 512


@functools.cache
def _make_sc_aggregate():
    mesh = plsc.VectorSubcoreMesh(core_axis_name="c", subcore_axis_name="s")

    @functools.partial(
        pl.kernel,
        mesh=mesh,
        compiler_params=pltpu.CompilerParams(use_tc_tiling_on_sc=False),
        out_type=jax.ShapeDtypeStruct((2, NPAD, DH), jnp.float32),
        scratch_types=[
            pltpu.VMEM((CHB + 1, K), jnp.int32),
            pltpu.VMEM((CHB + 1, K), jnp.int32),
            pltpu.VMEM((K, DH), jnp.float32),
            pltpu.VMEM((K, DH), jnp.float32),
            pltpu.VMEM((K, DH), jnp.float32),
            pltpu.VMEM_SHARED((NPAD, DH), jnp.float32),
            pltpu.SemaphoreType.DMA,
            pltpu.SemaphoreType.DMA,
        ],
    )
    def agg(h_hbm, ei_hbm, z_hbm, out_hbm, src_v, dst_v, rows0_v, rows1_v,
            rows2_v, acc_sh, gsem, ssem):
        cid = lax.axis_index("c")
        sid = lax.axis_index("s")
        r0 = sid * ROWS_PER_TILE
        pltpu.sync_copy(z_hbm.at[pl.ds(r0, ROWS_PER_TILE)],
                        acc_sh.at[pl.ds(r0, ROWS_PER_TILE)])
        pltpu.sync_copy(ei_hbm.at[0, pl.ds(sid * CHB, CHB)],
                        src_v.at[pl.ds(0, CHB)])
        pltpu.sync_copy(ei_hbm.at[1, pl.ds(sid * CHB, CHB)],
                        dst_v.at[pl.ds(0, CHB)])

        @pl.when(sid < NEXTRA)
        def _():
            pltpu.sync_copy(ei_hbm.at[0, pl.ds(NT * CHB + sid, 1)],
                            src_v.at[pl.ds(CHB, 1)])
            pltpu.sync_copy(ei_hbm.at[1, pl.ds(NT * CHB + sid, 1)],
                            dst_v.at[pl.ds(CHB, 1)])

        plsc.subcore_barrier()

        bufs = (rows0_v, rows1_v, rows2_v)

        def gather(j, buf):
            return pltpu.make_async_copy(h_hbm.at[cid].at[src_v.at[j]], buf, gsem)

        def scatter(j, buf):
            return pltpu.make_async_copy(buf, acc_sh.at[dst_v.at[j]], ssem)

        gather(0, rows0_v).start()
        gather(1, rows1_v).start()

        def body(t, carry):
            for r in range(3):
                j = 3 * t + r
                gather(j, bufs[r]).wait()

                @pl.when(j >= 1)
                def _():
                    scatter(j, bufs[r]).wait()

                @pl.when(j + 2 < CHB)
                def _():
                    gather(j + 2, bufs[(r + 2) % 3]).start()

                scatter(j, bufs[r]).start(add=True)
            return carry

        lax.fori_loop(0, CHB // 3, body, 0)
        scatter(0, rows0_v).wait()

        @pl.when(sid < NEXTRA)
        def _():
            gather(CHB, rows0_v).start()
            gather(CHB, rows0_v).wait()
            pltpu.sync_copy(rows0_v, acc_sh.at[dst_v.at[CHB]], add=True)

        plsc.subcore_barrier()
        pltpu.sync_copy(acc_sh.at[pl.ds(r0, ROWS_PER_TILE)],
                        out_hbm.at[cid, pl.ds(r0, ROWS_PER_TILE)])

    return agg


def _sc_aggregate(hsplit, ei, zeros):
    return _make_sc_aggregate()(hsplit, ei, zeros)


def _mean_and_h(acc_ref, h_ref):
    a = jnp.concatenate([acc_ref[0], acc_ref[1]], axis=1)
    inv = 1.0 / jnp.maximum(a[:, CNT_COL:CNT_COL + 1], 1.0)
    p = a * inv
    h = jnp.concatenate([h_ref[0], h_ref[1]], axis=1)
    return p, h


def _masked_linear(p, h, lm_ref, vlt, wlt, vut, wut, bl, bu):
    f32 = jnp.float32
    lab = (jnp.dot(h, vlt[...], preferred_element_type=f32)
           + jnp.dot(p, wlt[...], preferred_element_type=f32) + bl[...])
    unl = (jnp.dot(h, vut[...], preferred_element_type=f32)
           + jnp.dot(p, wut[...], preferred_element_type=f32) + bu[...])
    lm = lm_ref[...]
    return jnp.maximum(lm * lab + (1.0 - lm) * unl, 0.0)


def _tc_layer_body(acc_ref, h_ref, lm_ref, vlt, wlt, vut, wut, bl, bu, o_ref):
    p, h = _mean_and_h(acc_ref, h_ref)
    res = _masked_linear(p, h, lm_ref, vlt, wlt, vut, wut, bl, bu)
    o_ref[0] = res[:, :DH]
    o_ref[1] = res[:, DH:]


def _tc_final_body(acc_ref, h_ref, lm_ref, vlt, wlt, vut, wut, bl, bu,
                   cwt, cb, o_ref):
    p, h = _mean_and_h(acc_ref, h_ref)
    h1 = _masked_linear(p, h, lm_ref, vlt, wlt, vut, wut, bl, bu)
    logits = jnp.dot(h1, cwt[...], preferred_element_type=jnp.float32) + cb[...]
    m = jnp.max(logits, axis=1, keepdims=True)
    s = logits - m
    o_ref[...] = s - jnp.log(jnp.sum(jnp.exp(s), axis=1, keepdims=True))


def _prep_body(x_ref, o_ref):
    xb = x_ref[...]
    r = xb.shape[0]
    o_ref[0] = xb[:, :DH]
    o_ref[1] = jnp.concatenate(
        [xb[:, DH:DREF],
         jnp.ones((r, 1), jnp.float32),
         jnp.zeros((r, D - DREF - 1), jnp.float32)], axis=1)


def _tc_prep(x):
    grid = (NPAD // RBLK,)
    return pl.pallas_call(
        _prep_body,
        grid=grid,
        in_specs=[pl.BlockSpec((RBLK, DREF), lambda i: (i, 0))],
        out_specs=pl.BlockSpec((2, RBLK, DH), lambda i: (0, i, 0)),
        out_shape=jax.ShapeDtypeStruct((2, NPAD, DH), jnp.float32),
    )(x)


def _row_spec(r, c):
    return pl.BlockSpec((r, c), lambda i: (i, 0))


def _rep_spec(r, c):
    return pl.BlockSpec((r, c), lambda i: (0, 0))


_SPLIT_SPEC = pl.BlockSpec((2, RBLK, DH), lambda i: (0, i, 0))


def _tc_layer(acc, hsplit, lmf, vlt, wlt, vut, wut, bl, bu):
    grid = (NPAD // RBLK,)
    return pl.pallas_call(
        _tc_layer_body,
        grid=grid,
        in_specs=[
            _SPLIT_SPEC,
            _SPLIT_SPEC,
            _row_spec(RBLK, 1),
            _rep_spec(D, D), _rep_spec(D, D), _rep_spec(D, D), _rep_spec(D, D),
            _rep_spec(1, D), _rep_spec(1, D),
        ],
        out_specs=_SPLIT_SPEC,
        out_shape=jax.ShapeDtypeStruct((2, NPAD, DH), jnp.float32),
    )(acc, hsplit, lmf, vlt, wlt, vut, wut, bl, bu)


def _tc_final(acc, hsplit, lmf, vlt, wlt, vut, wut, bl, bu, cwt, cb):
    grid = (NPAD // RBLK,)
    nc = cwt.shape[1]
    return pl.pallas_call(
        _tc_final_body,
        grid=grid,
        in_specs=[
            _SPLIT_SPEC,
            _SPLIT_SPEC,
            _row_spec(RBLK, 1),
            _rep_spec(D, D), _rep_spec(D, D), _rep_spec(D, D), _rep_spec(D, D),
            _rep_spec(1, D), _rep_spec(1, D),
            _rep_spec(D, nc), _rep_spec(1, nc),
        ],
        out_specs=_row_spec(RBLK, nc),
        out_shape=jax.ShapeDtypeStruct((NPAD, nc), jnp.float32),
    )(acc, hsplit, lmf, vlt, wlt, vut, wut, bl, bu, cwt, cb)


def _pad_wt(w):
    return jnp.zeros((D, D), jnp.float32).at[:DREF, :DREF].set(w.T)


def _pad_bias(b1, b2):
    return jnp.zeros((1, D), jnp.float32).at[0, :DREF].set(b1 + b2).at[0, CNT_COL].set(1.0)


def kernel(x, edge_index, labelmask,
           l0_VLw, l0_VLb, l0_WLw, l0_WLb, l0_VUw, l0_VUb, l0_WUw, l0_WUb,
           l1_VLw, l1_VLb, l1_WLw, l1_WLb, l1_VUw, l1_VUb, l1_WUw, l1_WUb,
           Cw, Cb):
    hs0 = _tc_prep(x)
    ei3 = edge_index.reshape(2, NCH, K)
    zeros = jnp.zeros((NPAD, DH), jnp.float32)
    lmf = jnp.zeros((NPAD, 1), jnp.float32).at[:N, 0].set(labelmask.astype(jnp.float32))

    l0 = (_pad_wt(l0_VLw), _pad_wt(l0_WLw), _pad_wt(l0_VUw), _pad_wt(l0_WUw),
          _pad_bias(l0_VLb, l0_WLb), _pad_bias(l0_VUb, l0_WUb))
    l1 = (_pad_wt(l1_VLw), _pad_wt(l1_WLw), _pad_wt(l1_VUw), _pad_wt(l1_WUw),
          _pad_bias(l1_VLb, l1_WLb), _pad_bias(l1_VUb, l1_WUb))
    nc = Cw.shape[0]
    cwt = jnp.zeros((D, nc), jnp.float32).at[:DREF].set(Cw.T)
    cb = Cb.reshape(1, nc)

    acc0 = _sc_aggregate(hs0, ei3, zeros)
    hs1 = _tc_layer(acc0, hs0, lmf, *l0)
    acc1 = _sc_aggregate(hs1, ei3, zeros)
    out = _tc_final(acc1, hs1, lmf, *l1, cwt, cb)
    return out[:N]


# speedup vs baseline: 10.5265x; 1.0422x over previous
import functools

import jax
import jax.numpy as jnp
from jax import lax
from jax.experimental import pallas as pl
from jax.experimental.pallas import tpu as pltpu
from jax.experimental.pallas import tpu_sc as plsc

N = 10000
E = 320000
DREF = 145
D = 160
DH = D // 2
CNT_COL = 145
NPAD = 10240
NT = 16
K = 128
NCH = E // K
CHB = NCH // NT
NEXTRA = NCH - CHB * NT
ROWS_PER_TILE = NPAD // NT
RBLK = 1024


@functools.cache
def _make_sc_aggregate():
    mesh = plsc.VectorSubcoreMesh(core_axis_name="c", subcore_axis_name="s")

    @functools.partial(
        pl.kernel,
        mesh=mesh,
        compiler_params=pltpu.CompilerParams(use_tc_tiling_on_sc=False),
        out_type=jax.ShapeDtypeStruct((2, NPAD, DH), jnp.float32),
        scratch_types=[
            pltpu.VMEM((CHB + 1, K), jnp.int32),
            pltpu.VMEM((CHB + 1, K), jnp.int32),
            pltpu.VMEM((K, DH), jnp.float32),
            pltpu.VMEM((K, DH), jnp.float32),
            pltpu.VMEM((K, DH), jnp.float32),
            pltpu.VMEM_SHARED((NPAD, DH), jnp.float32),
            pltpu.SemaphoreType.DMA,
            pltpu.SemaphoreType.DMA,
        ],
    )
    def agg(h_hbm, ei_hbm, z_hbm, out_hbm, src_v, dst_v, rows0_v, rows1_v,
            rows2_v, acc_sh, gsem, ssem):
        cid = lax.axis_index("c")
        sid = lax.axis_index("s")
        r0 = sid * ROWS_PER_TILE
        pltpu.sync_copy(z_hbm.at[pl.ds(r0, ROWS_PER_TILE)],
                        acc_sh.at[pl.ds(r0, ROWS_PER_TILE)])
        pltpu.sync_copy(ei_hbm.at[0, pl.ds(sid * CHB, CHB)],
                        src_v.at[pl.ds(0, CHB)])
        pltpu.sync_copy(ei_hbm.at[1, pl.ds(sid * CHB, CHB)],
                        dst_v.at[pl.ds(0, CHB)])

        @pl.when(sid < NEXTRA)
        def _():
            pltpu.sync_copy(ei_hbm.at[0, pl.ds(NT * CHB + sid, 1)],
                            src_v.at[pl.ds(CHB, 1)])
            pltpu.sync_copy(ei_hbm.at[1, pl.ds(NT * CHB + sid, 1)],
                            dst_v.at[pl.ds(CHB, 1)])

        plsc.subcore_barrier()

        bufs = (rows0_v, rows1_v, rows2_v)

        def gather(j, buf):
            return pltpu.make_async_copy(h_hbm.at[cid].at[src_v.at[j]], buf, gsem)

        def scatter(j, buf):
            return pltpu.make_async_copy(buf, acc_sh.at[dst_v.at[j]], ssem)

        gather(0, rows0_v).start()
        gather(1, rows1_v).start()

        def body(t, carry):
            for r in range(3):
                j = 3 * t + r
                gather(j, bufs[r]).wait()

                @pl.when(j >= 1)
                def _():
                    scatter(j, bufs[r]).wait()

                @pl.when(j + 2 < CHB)
                def _():
                    gather(j + 2, bufs[(r + 2) % 3]).start()

                scatter(j, bufs[r]).start(add=True)
            return carry

        lax.fori_loop(0, CHB // 3, body, 0)
        scatter(0, rows0_v).wait()

        @pl.when(sid < NEXTRA)
        def _():
            gather(CHB, rows0_v).start()
            gather(CHB, rows0_v).wait()
            pltpu.sync_copy(rows0_v, acc_sh.at[dst_v.at[CHB]], add=True)

        plsc.subcore_barrier()
        pltpu.sync_copy(acc_sh.at[pl.ds(r0, ROWS_PER_TILE)],
                        out_hbm.at[cid, pl.ds(r0, ROWS_PER_TILE)])

    return agg


def _sc_aggregate(hsplit, ei, zeros):
    return _make_sc_aggregate()(hsplit, ei, zeros)


def _mean_and_h(acc_ref, h_ref):
    a = jnp.concatenate([acc_ref[0], acc_ref[1]], axis=1)
    inv = 1.0 / jnp.maximum(a[:, CNT_COL:CNT_COL + 1], 1.0)
    p = a * inv
    h = jnp.concatenate([h_ref[0], h_ref[1]], axis=1)
    return p, h


def _masked_linear(p, h, lm_ref, vlt, wlt, vut, wut, bl, bu):
    f32 = jnp.float32
    lab = (jnp.dot(h, vlt[...], preferred_element_type=f32)
           + jnp.dot(p, wlt[...], preferred_element_type=f32) + bl[...])
    unl = (jnp.dot(h, vut[...], preferred_element_type=f32)
           + jnp.dot(p, wut[...], preferred_element_type=f32) + bu[...])
    lm = lm_ref[...]
    return jnp.maximum(lm * lab + (1.0 - lm) * unl, 0.0)


def _tc_layer_body(acc_ref, h_ref, lm_ref, vlt, wlt, vut, wut, bl, bu, o_ref):
    p, h = _mean_and_h(acc_ref, h_ref)
    res = _masked_linear(p, h, lm_ref, vlt, wlt, vut, wut, bl, bu)
    o_ref[0] = res[:, :DH]
    o_ref[1] = res[:, DH:]


def _tc_final_body(acc_ref, h_ref, lm_ref, vlt, wlt, vut, wut, bl, bu,
                   cwt, cb, o_ref):
    p, h = _mean_and_h(acc_ref, h_ref)
    h1 = _masked_linear(p, h, lm_ref, vlt, wlt, vut, wut, bl, bu)
    logits = jnp.dot(h1, cwt[...], preferred_element_type=jnp.float32) + cb[...]
    m = jnp.max(logits, axis=1, keepdims=True)
    s = logits - m
    o_ref[...] = s - jnp.log(jnp.sum(jnp.exp(s), axis=1, keepdims=True))


def _prep_body(x_ref, o_ref):
    xb = x_ref[...]
    r = xb.shape[0]
    o_ref[0] = xb[:, :DH]
    o_ref[1] = jnp.concatenate(
        [xb[:, DH:DREF],
         jnp.ones((r, 1), jnp.float32),
         jnp.zeros((r, D - DREF - 1), jnp.float32)], axis=1)


def _tc_prep(x):
    grid = (NPAD // RBLK,)
    return pl.pallas_call(
        _prep_body,
        grid=grid,
        in_specs=[pl.BlockSpec((RBLK, DREF), lambda i: (i, 0))],
        out_specs=pl.BlockSpec((2, RBLK, DH), lambda i: (0, i, 0)),
        out_shape=jax.ShapeDtypeStruct((2, NPAD, DH), jnp.float32),
    )(x)


def _row_spec(r, c):
    return pl.BlockSpec((r, c), lambda i: (i, 0))


def _rep_spec(r, c):
    return pl.BlockSpec((r, c), lambda i: (0, 0))


_SPLIT_SPEC = pl.BlockSpec((2, RBLK, DH), lambda i: (0, i, 0))


def _tc_layer(acc, hsplit, lmf, vlt, wlt, vut, wut, bl, bu):
    grid = (NPAD // RBLK,)
    return pl.pallas_call(
        _tc_layer_body,
        grid=grid,
        in_specs=[
            _SPLIT_SPEC,
            _SPLIT_SPEC,
            _row_spec(RBLK, 1),
            _rep_spec(D, D), _rep_spec(D, D), _rep_spec(D, D), _rep_spec(D, D),
            _rep_spec(1, D), _rep_spec(1, D),
        ],
        out_specs=_SPLIT_SPEC,
        out_shape=jax.ShapeDtypeStruct((2, NPAD, DH), jnp.float32),
    )(acc, hsplit, lmf, vlt, wlt, vut, wut, bl, bu)


def _tc_final(acc, hsplit, lmf, vlt, wlt, vut, wut, bl, bu, cwt, cb):
    grid = (NPAD // RBLK,)
    nc = cwt.shape[1]
    return pl.pallas_call(
        _tc_final_body,
        grid=grid,
        in_specs=[
            _SPLIT_SPEC,
            _SPLIT_SPEC,
            _row_spec(RBLK, 1),
            _rep_spec(D, D), _rep_spec(D, D), _rep_spec(D, D), _rep_spec(D, D),
            _rep_spec(1, D), _rep_spec(1, D),
            _rep_spec(D, nc), _rep_spec(1, nc),
        ],
        out_specs=_row_spec(RBLK, nc),
        out_shape=jax.ShapeDtypeStruct((NPAD, nc), jnp.float32),
    )(acc, hsplit, lmf, vlt, wlt, vut, wut, bl, bu, cwt, cb)


def _pad_wt(w):
    return jnp.zeros((D, D), jnp.float32).at[:DREF, :DREF].set(w.T)


def _pad_bias(b1, b2):
    return jnp.zeros((1, D), jnp.float32).at[0, :DREF].set(b1 + b2).at[0, CNT_COL].set(1.0)


def kernel(x, edge_index, labelmask,
           l0_VLw, l0_VLb, l0_WLw, l0_WLb, l0_VUw, l0_VUb, l0_WUw, l0_WUb,
           l1_VLw, l1_VLb, l1_WLw, l1_WLb, l1_VUw, l1_VUb, l1_WUw, l1_WUb,
           Cw, Cb):
    hs0 = _tc_prep(x)
    ei3 = edge_index.reshape(2, NCH, K)
    zeros = jnp.zeros((NPAD, DH), jnp.float32)
    lmf = jnp.zeros((NPAD, 1), jnp.float32).at[:N, 0].set(labelmask.astype(jnp.float32))

    l0 = (_pad_wt(l0_VLw), _pad_wt(l0_WLw), _pad_wt(l0_VUw), _pad_wt(l0_WUw),
          _pad_bias(l0_VLb, l0_WLb), _pad_bias(l0_VUb, l0_WUb))
    l1 = (_pad_wt(l1_VLw), _pad_wt(l1_WLw), _pad_wt(l1_VUw), _pad_wt(l1_WUw),
          _pad_bias(l1_VLb, l1_WLb), _pad_bias(l1_VUb, l1_WUb))
    nc = Cw.shape[0]
    cwt = jnp.zeros((D, nc), jnp.float32).at[:DREF].set(Cw.T)
    cb = Cb.reshape(1, nc)

    acc0 = _sc_aggregate(hs0, ei3, zeros)
    hs1 = _tc_layer(acc0, hs0, lmf, *l0)
    acc1 = _sc_aggregate(hs1, ei3, zeros)
    out = _tc_final(acc1, hs1, lmf, *l1, cwt, cb)
    return out[:N]


# speedup vs baseline: 10.5527x; 1.0025x over previous
import functools

import jax
import jax.numpy as jnp
from jax import lax
from jax.experimental import pallas as pl
from jax.experimental.pallas import tpu as pltpu
from jax.experimental.pallas import tpu_sc as plsc

N = 10000
E = 320000
DREF = 145
D = 160
DH = D // 2
CNT_COL = 145
NPAD = 10240
NT = 16
K = 128
NCH = E // K
CHB = NCH // NT
NEXTRA = NCH - CHB * NT
ROWS_PER_TILE = NPAD // NT
RBLK = 1024


@functools.cache
def _make_sc_aggregate():
    mesh = plsc.VectorSubcoreMesh(core_axis_name="c", subcore_axis_name="s")

    @functools.partial(
        pl.kernel,
        mesh=mesh,
        compiler_params=pltpu.CompilerParams(use_tc_tiling_on_sc=False),
        out_type=jax.ShapeDtypeStruct((2, NPAD, DH), jnp.float32),
        scratch_types=[
            pltpu.VMEM((CHB + 1, K), jnp.int32),
            pltpu.VMEM((CHB + 1, K), jnp.int32),
            pltpu.VMEM((K, DH), jnp.float32),
            pltpu.VMEM((K, DH), jnp.float32),
            pltpu.VMEM((K, DH), jnp.float32),
            pltpu.VMEM_SHARED((NPAD, DH), jnp.float32),
            pltpu.SemaphoreType.DMA,
            pltpu.SemaphoreType.DMA,
        ],
    )
    def agg(h_hbm, ei_hbm, z_hbm, out_hbm, src_v, dst_v, rows0_v, rows1_v,
            rows2_v, acc_sh, gsem, ssem):
        cid = lax.axis_index("c")
        sid = lax.axis_index("s")
        r0 = sid * ROWS_PER_TILE
        pltpu.sync_copy(z_hbm.at[pl.ds(r0, ROWS_PER_TILE)],
                        acc_sh.at[pl.ds(r0, ROWS_PER_TILE)])
        pltpu.sync_copy(ei_hbm.at[0, pl.ds(sid * CHB, CHB)],
                        src_v.at[pl.ds(0, CHB)])
        pltpu.sync_copy(ei_hbm.at[1, pl.ds(sid * CHB, CHB)],
                        dst_v.at[pl.ds(0, CHB)])

        @pl.when(sid < NEXTRA)
        def _():
            pltpu.sync_copy(ei_hbm.at[0, pl.ds(NT * CHB + sid, 1)],
                            src_v.at[pl.ds(CHB, 1)])
            pltpu.sync_copy(ei_hbm.at[1, pl.ds(NT * CHB + sid, 1)],
                            dst_v.at[pl.ds(CHB, 1)])

        plsc.subcore_barrier()

        bufs = (rows0_v, rows1_v, rows2_v)

        def gather(j, buf):
            return pltpu.make_async_copy(h_hbm.at[cid].at[src_v.at[j]], buf, gsem)

        def scatter(j, buf):
            return pltpu.make_async_copy(buf, acc_sh.at[dst_v.at[j]], ssem)

        gather(0, rows0_v).start()
        gather(1, rows1_v).start()

        def body(t, carry):
            for r in range(3):
                j = 3 * t + r
                gather(j, bufs[r]).wait()

                @pl.when(j >= 1)
                def _():
                    scatter(j, bufs[r]).wait()

                @pl.when(j + 2 < CHB)
                def _():
                    gather(j + 2, bufs[(r + 2) % 3]).start()

                scatter(j, bufs[r]).start(add=True)
            return carry

        lax.fori_loop(0, CHB // 3, body, 0)
        scatter(0, rows0_v).wait()

        @pl.when(sid < NEXTRA)
        def _():
            gather(CHB, rows0_v).start()
            gather(CHB, rows0_v).wait()
            pltpu.sync_copy(rows0_v, acc_sh.at[dst_v.at[CHB]], add=True)

        plsc.subcore_barrier()
        pltpu.sync_copy(acc_sh.at[pl.ds(r0, ROWS_PER_TILE)],
                        out_hbm.at[cid, pl.ds(r0, ROWS_PER_TILE)])

    return agg


def _sc_aggregate(hsplit, ei, zeros):
    return _make_sc_aggregate()(hsplit, ei, zeros)


def _mean_and_h(acc_ref, h_ref):
    a = jnp.concatenate([acc_ref[0], acc_ref[1]], axis=1)
    inv = 1.0 / jnp.maximum(a[:, CNT_COL:CNT_COL + 1], 1.0)
    p = a * inv
    h = jnp.concatenate([h_ref[0], h_ref[1]], axis=1)
    return p, h


def _masked_linear(p, h, lm_ref, vlt, wlt, vut, wut, bl, bu):
    f32 = jnp.float32
    lab = (jnp.dot(h, vlt[...], preferred_element_type=f32)
           + jnp.dot(p, wlt[...], preferred_element_type=f32) + bl[...])
    unl = (jnp.dot(h, vut[...], preferred_element_type=f32)
           + jnp.dot(p, wut[...], preferred_element_type=f32) + bu[...])
    lm = lm_ref[...]
    return jnp.maximum(lm * lab + (1.0 - lm) * unl, 0.0)


def _tc_layer_body(acc_ref, h_ref, lm_ref, vlt, wlt, vut, wut, bl, bu, o_ref):
    p, h = _mean_and_h(acc_ref, h_ref)
    res = _masked_linear(p, h, lm_ref, vlt, wlt, vut, wut, bl, bu)
    o_ref[0] = res[:, :DH]
    o_ref[1] = res[:, DH:]


def _tc_final_body(acc_ref, h_ref, lm_ref, vlt, wlt, vut, wut, bl, bu,
                   cwt, cb, o_ref):
    p, h = _mean_and_h(acc_ref, h_ref)
    h1 = _masked_linear(p, h, lm_ref, vlt, wlt, vut, wut, bl, bu)
    logits = jnp.dot(h1, cwt[...], preferred_element_type=jnp.float32) + cb[...]
    m = jnp.max(logits, axis=1, keepdims=True)
    s = logits - m
    o_ref[...] = s - jnp.log(jnp.sum(jnp.exp(s), axis=1, keepdims=True))


def _prep_body(x_ref, o_ref):
    xb = x_ref[...]
    r = xb.shape[0]
    o_ref[0] = xb[:, :DH]
    o_ref[1] = jnp.concatenate(
        [xb[:, DH:DREF],
         jnp.ones((r, 1), jnp.float32),
         jnp.zeros((r, D - DREF - 1), jnp.float32)], axis=1)


def _tc_prep(x):
    grid = (NPAD // RBLK,)
    return pl.pallas_call(
        _prep_body,
        grid=grid,
        in_specs=[pl.BlockSpec((RBLK, DREF), lambda i: (i, 0))],
        out_specs=pl.BlockSpec((2, RBLK, DH), lambda i: (0, i, 0)),
        out_shape=jax.ShapeDtypeStruct((2, NPAD, DH), jnp.float32),
    )(x)


def _row_spec(r, c):
    return pl.BlockSpec((r, c), lambda i: (i, 0))


def _rep_spec(r, c):
    return pl.BlockSpec((r, c), lambda i: (0, 0))


_SPLIT_SPEC = pl.BlockSpec((2, RBLK, DH), lambda i: (0, i, 0))


def _tc_layer(acc, hsplit, lmf, vlt, wlt, vut, wut, bl, bu):
    grid = (NPAD // RBLK,)
    return pl.pallas_call(
        _tc_layer_body,
        grid=grid,
        in_specs=[
            _SPLIT_SPEC,
            _SPLIT_SPEC,
            _row_spec(RBLK, 1),
            _rep_spec(D, D), _rep_spec(D, D), _rep_spec(D, D), _rep_spec(D, D),
            _rep_spec(1, D), _rep_spec(1, D),
        ],
        out_specs=_SPLIT_SPEC,
        out_shape=jax.ShapeDtypeStruct((2, NPAD, DH), jnp.float32),
    )(acc, hsplit, lmf, vlt, wlt, vut, wut, bl, bu)


FBLK = 1000


def _tc_final(acc, hsplit, lmf, vlt, wlt, vut, wut, bl, bu, cwt, cb):
    grid = (N // FBLK,)
    nc = cwt.shape[1]
    return pl.pallas_call(
        _tc_final_body,
        grid=grid,
        in_specs=[
            pl.BlockSpec((2, FBLK, DH), lambda i: (0, i, 0)),
            pl.BlockSpec((2, FBLK, DH), lambda i: (0, i, 0)),
            _row_spec(FBLK, 1),
            _rep_spec(D, D), _rep_spec(D, D), _rep_spec(D, D), _rep_spec(D, D),
            _rep_spec(1, D), _rep_spec(1, D),
            _rep_spec(D, nc), _rep_spec(1, nc),
        ],
        out_specs=_row_spec(FBLK, nc),
        out_shape=jax.ShapeDtypeStruct((N, nc), jnp.float32),
    )(acc, hsplit, lmf, vlt, wlt, vut, wut, bl, bu, cwt, cb)


def _pad_wt(w):
    return jnp.zeros((D, D), jnp.float32).at[:DREF, :DREF].set(w.T)


def _pad_bias(b1, b2):
    return jnp.zeros((1, D), jnp.float32).at[0, :DREF].set(b1 + b2).at[0, CNT_COL].set(1.0)


def kernel(x, edge_index, labelmask,
           l0_VLw, l0_VLb, l0_WLw, l0_WLb, l0_VUw, l0_VUb, l0_WUw, l0_WUb,
           l1_VLw, l1_VLb, l1_WLw, l1_WLb, l1_VUw, l1_VUb, l1_WUw, l1_WUb,
           Cw, Cb):
    hs0 = _tc_prep(x)
    ei3 = edge_index.reshape(2, NCH, K)
    zeros = jnp.zeros((NPAD, DH), jnp.float32)
    lmf = jnp.zeros((NPAD, 1), jnp.float32).at[:N, 0].set(labelmask.astype(jnp.float32))

    l0 = (_pad_wt(l0_VLw), _pad_wt(l0_WLw), _pad_wt(l0_VUw), _pad_wt(l0_WUw),
          _pad_bias(l0_VLb, l0_WLb), _pad_bias(l0_VUb, l0_WUb))
    l1 = (_pad_wt(l1_VLw), _pad_wt(l1_WLw), _pad_wt(l1_VUw), _pad_wt(l1_WUw),
          _pad_bias(l1_VLb, l1_WLb), _pad_bias(l1_VUb, l1_WUb))
    nc = Cw.shape[0]
    cwt = jnp.zeros((D, nc), jnp.float32).at[:DREF].set(Cw.T)
    cb = Cb.reshape(1, nc)

    acc0 = _sc_aggregate(hs0, ei3, zeros)
    hs1 = _tc_layer(acc0, hs0, lmf, *l0)
    acc1 = _sc_aggregate(hs1, ei3, zeros)
    return _tc_final(acc1, hs1, lmf, *l1, cwt, cb)
